# Initial kernel scaffold; baseline (speedup 1.0000x reference)
import functools

import jax
import jax.numpy as jnp
from jax import lax
from jax.experimental import pallas as pl
from jax.experimental.pallas import tpu as pltpu
from jax.experimental.pallas import tpu_sc as plsc

ROWS, COLS = 1024, 32768
NC, NS, L = 2, 16, 16
NW = NC * NS
ROWS_PER_W = ROWS // NW

_MESH = plsc.VectorSubcoreMesh(
    core_axis_name="c", subcore_axis_name="s", num_cores=NC, num_subcores=NS
)


@functools.partial(
    pl.kernel,
    out_type=jax.ShapeDtypeStruct((ROWS, COLS), jnp.float32),
    mesh=_MESH,
    scratch_types=[
        pltpu.VMEM((COLS,), jnp.float32),
        pltpu.VMEM((COLS,), jnp.float32),
    ],
)
def _masked_cumsum_sc(x_hbm, m_hbm, out_hbm, xbuf, mbuf):
    wid = lax.axis_index("s") * NC + lax.axis_index("c")
    last = jnp.full((L,), L - 1, jnp.int32)

    def do_row(r, _):
        row = wid * ROWS_PER_W + r
        pltpu.sync_copy(x_hbm.at[row], xbuf)
        pltpu.sync_copy(m_hbm.at[row], mbuf)

        def do_vec(j, carry):
            sl = pl.ds(j * L, L)
            v = xbuf[sl] * mbuf[sl]
            s = plsc.cumsum(v) + carry
            xbuf[sl] = s
            return jnp.take(s, last, mode="promise_in_bounds")

        lax.fori_loop(0, COLS // L, do_vec, jnp.zeros((L,), jnp.float32),
                      unroll=8)
        pltpu.sync_copy(xbuf, out_hbm.at[row])
        return 0

    lax.fori_loop(0, ROWS_PER_W, do_row, 0)


def kernel(x, mask):
    return _masked_cumsum_sc(x, mask.astype(jnp.float32))


# speedup vs baseline: 1.1926x; 1.1926x over previous
import functools

import jax
import jax.numpy as jnp
from jax import lax
from jax.experimental import pallas as pl
from jax.experimental.pallas import tpu as pltpu
from jax.experimental.pallas import tpu_sc as plsc

ROWS, COLS = 1024, 32768
NC, NS, L = 2, 16, 16
NW = NC * NS
ROWS_PER_W = ROWS // NW

_MESH = plsc.VectorSubcoreMesh(
    core_axis_name="c", subcore_axis_name="s", num_cores=NC, num_subcores=NS
)


@functools.partial(
    pl.kernel,
    out_type=jax.ShapeDtypeStruct((ROWS, COLS), jnp.float32),
    mesh=_MESH,
    scratch_types=[
        pltpu.VMEM((COLS,), jnp.float32),
        pltpu.VMEM((COLS,), jnp.float32),
    ],
    compiler_params=pltpu.CompilerParams(needs_layout_passes=False),
)
def _masked_cumsum_sc(x_hbm, m_hbm, out_hbm, xbuf, mbuf):
    wid = lax.axis_index("s") * NC + lax.axis_index("c")
    last = jnp.full((L,), L - 1, jnp.int32)

    def do_row(r, _):
        row = wid * ROWS_PER_W + r
        pltpu.sync_copy(x_hbm.at[row], xbuf)
        pltpu.sync_copy(m_hbm.at[row], mbuf)

        def do_vec(j, carry):
            sl = pl.ds(j * L, L)
            v = xbuf[sl] * mbuf[sl]
            s = plsc.cumsum(v) + carry
            xbuf[sl] = s
            return jnp.take_along_axis(s, last, axis=0,
                                       mode="promise_in_bounds")

        lax.fori_loop(0, COLS // L, do_vec, jnp.zeros((L,), jnp.float32),
                      unroll=8)
        pltpu.sync_copy(xbuf, out_hbm.at[row])
        return 0

    lax.fori_loop(0, ROWS_PER_W, do_row, 0)


def kernel(x, mask):
    return _masked_cumsum_sc(x, mask.astype(jnp.float32))


# speedup vs baseline: 2.0188x; 1.6928x over previous
import functools

import jax
import jax.numpy as jnp
from jax import lax
from jax.experimental import pallas as pl
from jax.experimental.pallas import tpu as pltpu
from jax.experimental.pallas import tpu_sc as plsc

ROWS, COLS = 1024, 32768
NC, NS, L = 2, 16, 16
NW = NC * NS
ROWS_PER_W = ROWS // NW
G = 8
NGRP = ROWS_PER_W // G
CHUNK = 2048
NCH = COLS // CHUNK

_MESH = plsc.VectorSubcoreMesh(
    core_axis_name="c", subcore_axis_name="s", num_cores=NC, num_subcores=NS
)


@functools.partial(
    pl.kernel,
    out_type=jax.ShapeDtypeStruct((ROWS, COLS), jnp.float32),
    mesh=_MESH,
    scratch_types=[
        pltpu.VMEM((G, CHUNK), jnp.float32),
        pltpu.VMEM((G, CHUNK), jnp.float32),
        pltpu.SemaphoreType.DMA,
        pltpu.SemaphoreType.DMA,
    ],
    compiler_params=pltpu.CompilerParams(needs_layout_passes=False),
)
def _masked_cumsum_sc(x_hbm, m_hbm, out_hbm, xbuf, mbuf, sem_in, sem_out):
    wid = lax.axis_index("s") * NC + lax.axis_index("c")
    base_row = wid * ROWS_PER_W
    last = jnp.full((L,), L - 1, jnp.int32)

    def splat_last(s):
        return jnp.take_along_axis(s, last, axis=0, mode="promise_in_bounds")

    def do_group(grp, _):
        row0 = base_row + grp * G

        def do_chunk(c, carries):
            c0 = c * CHUNK

            @pl.when(c > 0)
            def _():
                for g in range(G):
                    pltpu.make_async_copy(
                        xbuf.at[g], out_hbm.at[row0 + g, pl.ds(0, CHUNK)],
                        sem_out,
                    ).wait()

            copies = []
            for g in range(G):
                copies.append(pltpu.async_copy(
                    x_hbm.at[row0 + g, pl.ds(c0, CHUNK)], xbuf.at[g], sem_in))
                copies.append(pltpu.async_copy(
                    m_hbm.at[row0 + g, pl.ds(c0, CHUNK)], mbuf.at[g], sem_in))
            for cp in copies:
                cp.wait()

            def do_vec(j, cs):
                sl = pl.ds(j * L, L)
                out = []
                for g in range(G):
                    v = xbuf[g, sl] * mbuf[g, sl]
                    s = plsc.cumsum(v) + cs[g]
                    xbuf[g, sl] = s
                    out.append(splat_last(s))
                return tuple(out)

            carries = lax.fori_loop(0, CHUNK // L, do_vec, carries, unroll=2)

            for g in range(G):
                pltpu.async_copy(
                    xbuf.at[g], out_hbm.at[row0 + g, pl.ds(c0, CHUNK)],
                    sem_out)
            return carries

        zeros = tuple(jnp.zeros((L,), jnp.float32) for _ in range(G))
        lax.fori_loop(0, NCH, do_chunk, zeros)

        for g in range(G):
            pltpu.make_async_copy(
                xbuf.at[g], out_hbm.at[row0 + g, pl.ds(0, CHUNK)], sem_out,
            ).wait()
        return 0

    lax.fori_loop(0, NGRP, do_group, 0)


def kernel(x, mask):
    return _masked_cumsum_sc(x, mask.astype(jnp.float32))


# speedup vs baseline: 2.2609x; 1.1200x over previous
import functools

import jax
import jax.numpy as jnp
from jax import lax
from jax.experimental import pallas as pl
from jax.experimental.pallas import tpu as pltpu
from jax.experimental.pallas import tpu_sc as plsc

ROWS, COLS = 1024, 32768
NC, NS, L = 2, 16, 16
NW = NC * NS
ROWS_PER_W = ROWS // NW
G = 8
NGRP = ROWS_PER_W // G
CHUNK = 2048
NCH = COLS // CHUNK

_MESH = plsc.VectorSubcoreMesh(
    core_axis_name="c", subcore_axis_name="s", num_cores=NC, num_subcores=NS
)


@functools.partial(
    pl.kernel,
    out_type=jax.ShapeDtypeStruct((ROWS, COLS), jnp.float32),
    mesh=_MESH,
    scratch_types=[
        pltpu.VMEM((2, G, CHUNK), jnp.float32),
        pltpu.VMEM((2, G, CHUNK), jnp.float32),
        pltpu.SemaphoreType.DMA,
        pltpu.SemaphoreType.DMA,
        pltpu.SemaphoreType.DMA,
    ],
    compiler_params=pltpu.CompilerParams(needs_layout_passes=False),
)
def _masked_cumsum_sc(x_hbm, m_hbm, out_hbm, xbuf, mbuf, sem0, sem1, sem_out):
    wid = lax.axis_index("s") * NC + lax.axis_index("c")
    base_row = wid * ROWS_PER_W
    last = jnp.full((L,), L - 1, jnp.int32)
    sems = (sem0, sem1)

    def splat_last(s):
        return jnp.take_along_axis(s, last, axis=0, mode="promise_in_bounds")

    def do_group(grp, _):
        row0 = base_row + grp * G

        def issue_inputs(slot, c):
            c0 = c * CHUNK
            for g in range(G):
                pltpu.async_copy(
                    x_hbm.at[row0 + g, pl.ds(c0, CHUNK)],
                    xbuf.at[slot, g], sems[slot])
                pltpu.async_copy(
                    m_hbm.at[row0 + g, pl.ds(c0, CHUNK)],
                    mbuf.at[slot, g], sems[slot])

        def wait_inputs(slot):
            for g in range(G):
                pltpu.make_async_copy(
                    x_hbm.at[row0 + g, pl.ds(0, CHUNK)],
                    xbuf.at[slot, g], sems[slot]).wait()
                pltpu.make_async_copy(
                    m_hbm.at[row0 + g, pl.ds(0, CHUNK)],
                    mbuf.at[slot, g], sems[slot]).wait()

        def drain_outputs(slot):
            for g in range(G):
                pltpu.make_async_copy(
                    xbuf.at[slot, g],
                    out_hbm.at[row0 + g, pl.ds(0, CHUNK)], sem_out).wait()

        issue_inputs(0, 0)

        def do_pair(cc, carries):
            for par in range(2):
                c = cc * 2 + par
                slot, other = par, 1 - par

                @pl.when(c < NCH - 1)
                def _():
                    @pl.when(c >= 1)
                    def _():
                        drain_outputs(other)
                    issue_inputs(other, c + 1)

                wait_inputs(slot)

                def do_vec(j, cs):
                    sl = pl.ds(j * L, L)
                    out = []
                    for g in range(G):
                        v = xbuf[slot, g, sl] * mbuf[slot, g, sl]
                        s = plsc.cumsum(v) + cs[g]
                        xbuf[slot, g, sl] = s
                        out.append(splat_last(s))
                    return tuple(out)

                carries = lax.fori_loop(0, CHUNK // L, do_vec, carries,
                                        unroll=2)

                c0 = c * CHUNK
                for g in range(G):
                    pltpu.async_copy(
                        xbuf.at[slot, g],
                        out_hbm.at[row0 + g, pl.ds(c0, CHUNK)], sem_out)
            return carries

        zeros = tuple(jnp.zeros((L,), jnp.float32) for _ in range(G))
        lax.fori_loop(0, NCH // 2, do_pair, zeros)

        drain_outputs(0)
        drain_outputs(1)
        return 0

    lax.fori_loop(0, NGRP, do_group, 0)


def kernel(x, mask):
    return _masked_cumsum_sc(x, mask.astype(jnp.float32))
